# Initial kernel scaffold; baseline (speedup 1.0000x reference)
import jax
import jax.numpy as jnp
from jax.experimental import pallas as pl


def kernel(x, embedding_list):
    raise NotImplementedError("write your pallas kernel here")


# speedup vs baseline: 3.8814x; 3.8814x over previous
import functools

import jax
import jax.numpy as jnp
from jax import lax
from jax.experimental import pallas as pl
from jax.experimental.pallas import tpu as pltpu
from jax.experimental.pallas import tpu_sc as plsc

N_ATOMS = 1048576
N_EMB = 128

NC = 2
NS = 16
NW = NC * NS

G = N_ATOMS // 128
GPW = G // NW
K = 4
STEPS = GPW // K


def _sc_gather(table, x2d):
    mesh = plsc.VectorSubcoreMesh(core_axis_name="c", subcore_axis_name="s")

    @functools.partial(
        pl.kernel,
        mesh=mesh,
        out_type=jax.ShapeDtypeStruct((G, 128, N_EMB), jnp.float32),
        scratch_types=[
            pltpu.VMEM((K, 128), jnp.int32),
            pltpu.VMEM((K, 128, N_EMB), jnp.float32),
            pltpu.SemaphoreType.DMA,
        ],
    )
    def body(table_hbm, idx_hbm, out_hbm, idx_v, rows_v, sem):
        wid = lax.axis_index("s") * NC + lax.axis_index("c")

        def step(t, carry):
            g = wid * GPW + t * K
            pltpu.sync_copy(idx_hbm.at[pl.ds(g, K)], idx_v)
            copies = [
                pltpu.async_copy(table_hbm.at[idx_v.at[j]], rows_v.at[j], sem)
                for j in range(K)
            ]
            for c in copies:
                c.wait()
            pltpu.sync_copy(rows_v, out_hbm.at[pl.ds(g, K)])
            return carry

        lax.fori_loop(0, STEPS, step, 0)

    return body(table, x2d)


def kernel(x, embedding_list):
    out = _sc_gather(embedding_list, x.reshape(G, 128))
    return out.reshape(N_ATOMS, N_EMB)


# speedup vs baseline: 3.8969x; 1.0040x over previous
import functools

import jax
import jax.numpy as jnp
from jax import lax
from jax.experimental import pallas as pl
from jax.experimental.pallas import tpu as pltpu
from jax.experimental.pallas import tpu_sc as plsc

N_ATOMS = 1048576
N_EMB = 128

NC = 2
NS = 16
NW = NC * NS

G = N_ATOMS // 128
GPW = G // NW
NBUF = 4
UNROLL = NBUF


def _sc_gather(table, x2d):
    mesh = plsc.VectorSubcoreMesh(core_axis_name="c", subcore_axis_name="s")

    @functools.partial(
        pl.kernel,
        mesh=mesh,
        out_type=jax.ShapeDtypeStruct((G, 128, N_EMB), jnp.float32),
        scratch_types=[
            pltpu.VMEM((GPW, 128), jnp.int32),
            pltpu.VMEM((NBUF, 128, N_EMB), jnp.float32),
            pltpu.SemaphoreType.DMA,
            pltpu.SemaphoreType.DMA,
        ],
    )
    def body(table_hbm, idx_hbm, out_hbm, idx_v, rows_v, sem_g, sem_s):
        wid = lax.axis_index("s") * NC + lax.axis_index("c")
        base = wid * GPW

        def gather(t, b):
            pltpu.async_copy(table_hbm.at[idx_v.at[t]], rows_v.at[b], sem_g)

        def gather_wait(t, b):
            pltpu.make_async_copy(table_hbm.at[idx_v.at[t]], rows_v.at[b], sem_g).wait()

        def scatter(t, b):
            pltpu.async_copy(rows_v.at[b], out_hbm.at[base + t], sem_s)

        def scatter_wait(t, b):
            pltpu.make_async_copy(rows_v.at[b], out_hbm.at[base + t], sem_s).wait()

        pltpu.sync_copy(idx_hbm.at[pl.ds(base, GPW)], idx_v)

        gather(0, 0)
        gather(1, 1)

        def step(i, carry):
            for u in range(UNROLL):
                t = i * UNROLL + u
                gather_wait(t, u)
                scatter(t, u)

                @pl.when(t >= 2)
                def _():
                    scatter_wait(t - 2, (u + 2) % NBUF)

                @pl.when(t + 2 < GPW)
                def _():
                    gather(t + 2, (u + 2) % NBUF)
            return carry

        lax.fori_loop(0, GPW // UNROLL, step, 0)

        scatter_wait(GPW - 2, (GPW - 2) % NBUF)
        scatter_wait(GPW - 1, (GPW - 1) % NBUF)

    return body(table, x2d)


def kernel(x, embedding_list):
    out = _sc_gather(embedding_list, x.reshape(G, 128))
    return out.reshape(N_ATOMS, N_EMB)


# speedup vs baseline: 20.1766x; 5.1777x over previous
import functools

import jax
import jax.numpy as jnp
from jax import lax
from jax.experimental import pallas as pl
from jax.experimental.pallas import tpu as pltpu
from jax.experimental.pallas import tpu_sc as plsc

N_ATOMS = 1048576
N_EMB = 128

NC = 2
NS = 16
NW = NC * NS

G = N_ATOMS // 128
GPW = G // NW
NBUF = 4
UNROLL = NBUF


def _sc_gather(table, x2d):
    mesh = plsc.VectorSubcoreMesh(core_axis_name="c", subcore_axis_name="s")

    @functools.partial(
        pl.kernel,
        mesh=mesh,
        out_type=jax.ShapeDtypeStruct((G, 128, N_EMB), jnp.float32),
        scratch_types=[
            pltpu.VMEM((GPW, 128), jnp.int32),
            pltpu.VMEM((NBUF, 128, N_EMB), jnp.float32),
            pltpu.VMEM_SHARED((100, N_EMB), jnp.float32),
            pltpu.SemaphoreType.DMA,
            pltpu.SemaphoreType.DMA,
        ],
    )
    def body(table_hbm, idx_hbm, out_hbm, idx_v, rows_v, table_sh, sem_g, sem_s):
        wid = lax.axis_index("s") * NC + lax.axis_index("c")
        base = wid * GPW

        @pl.when(lax.axis_index("s") == 0)
        def _():
            pltpu.sync_copy(table_hbm, table_sh)

        plsc.subcore_barrier()

        def gather(t, b):
            pltpu.async_copy(table_sh.at[idx_v.at[t]], rows_v.at[b], sem_g)

        def gather_wait(t, b):
            pltpu.make_async_copy(table_sh.at[idx_v.at[t]], rows_v.at[b], sem_g).wait()

        def scatter(t, b):
            pltpu.async_copy(rows_v.at[b], out_hbm.at[base + t], sem_s)

        def scatter_wait(t, b):
            pltpu.make_async_copy(rows_v.at[b], out_hbm.at[base + t], sem_s).wait()

        pltpu.sync_copy(idx_hbm.at[pl.ds(base, GPW)], idx_v)

        gather(0, 0)
        gather(1, 1)

        def step(i, carry):
            for u in range(UNROLL):
                t = i * UNROLL + u
                gather_wait(t, u)
                scatter(t, u)

                @pl.when(t >= 2)
                def _():
                    scatter_wait(t - 2, (u + 2) % NBUF)

                @pl.when(t + 2 < GPW)
                def _():
                    gather(t + 2, (u + 2) % NBUF)
            return carry

        lax.fori_loop(0, GPW // UNROLL, step, 0)

        scatter_wait(GPW - 2, (GPW - 2) % NBUF)
        scatter_wait(GPW - 1, (GPW - 1) % NBUF)

    return body(table, x2d)


def kernel(x, embedding_list):
    out = _sc_gather(embedding_list, x.reshape(G, 128))
    return out.reshape(N_ATOMS, N_EMB)
